# baseline (speedup 1.0000x reference)
import jax
import jax.numpy as jnp
from jax.experimental import pallas as pl

N = 100000
C_IN = 128
C_OUT = 64


def _bn_relu_body(x_ref, m_ref, s_ref, g_ref, b_ref, o_ref):
    x = x_ref[...]
    o_ref[...] = jnp.maximum((x - m_ref[...]) * s_ref[...] * g_ref[...] + b_ref[...], 0.0)


def _bn_relu(x, g, b):
    m = x.mean(axis=0, keepdims=True)
    v = x.var(axis=0, keepdims=True)
    s = 1.0 / jnp.sqrt(v + 1e-5)
    n, c = x.shape
    blk = 1000
    return pl.pallas_call(
        _bn_relu_body,
        grid=(n // blk,),
        in_specs=[
            pl.BlockSpec((blk, c), lambda i: (i, 0)),
            pl.BlockSpec((1, c), lambda i: (0, 0)),
            pl.BlockSpec((1, c), lambda i: (0, 0)),
            pl.BlockSpec((1, c), lambda i: (0, 0)),
            pl.BlockSpec((1, c), lambda i: (0, 0)),
        ],
        out_specs=pl.BlockSpec((blk, c), lambda i: (i, 0)),
        out_shape=jax.ShapeDtypeStruct((n, c), x.dtype),
    )(x, m, s, g.reshape(1, c), b.reshape(1, c))


def _spconv(x, W, src, dst, n):
    out = jnp.zeros((n, W.shape[-1]), dtype=x.dtype)
    for k in range(W.shape[0]):
        msg = x[src[k]] @ W[k]
        out = out.at[dst[k]].add(msg)
    return out


def kernel(feats, up_feats, inv_src, inv_dst, sub_src, sub_dst, g_up, b_up, W_inv,
           g1_0, b1_0, W1_0, g2_0, b2_0, W2_0, Wres0,
           g1_1, b1_1, W1_1, g2_1, b2_1, W2_1):
    x = _bn_relu(feats, g_up, b_up)
    up = _spconv(x, W_inv, inv_src, inv_dst, N)
    h = jnp.concatenate([up_feats, up], axis=1)
    t = _bn_relu(h, g1_0, b1_0)
    t = _spconv(t, W1_0, sub_src, sub_dst, N)
    t = _bn_relu(t, g2_0, b2_0)
    t = _spconv(t, W2_0, sub_src, sub_dst, N)
    h = t + h @ Wres0
    t = _bn_relu(h, g1_1, b1_1)
    t = _spconv(t, W1_1, sub_src, sub_dst, N)
    t = _bn_relu(t, g2_1, b2_1)
    t = _spconv(t, W2_1, sub_src, sub_dst, N)
    return h + t


# speedup vs baseline: 1.4145x; 1.4145x over previous
import functools

import jax
import jax.numpy as jnp
from jax import lax
from jax.experimental import pallas as pl
from jax.experimental.pallas import tpu as pltpu
from jax.experimental.pallas import tpu_sc as plsc

N = 100000
C_IN = 128
C_OUT = 64
KI = 8
KS = 27


def _stats_body(x_ref, s_ref, q_ref):
    @pl.when(pl.program_id(0) == 0)
    def _init():
        s_ref[...] = jnp.zeros_like(s_ref)
        q_ref[...] = jnp.zeros_like(q_ref)

    x = x_ref[...]
    s_ref[...] += jnp.sum(x, axis=0, keepdims=True)
    q_ref[...] += jnp.sum(x * x, axis=0, keepdims=True)


def _stats(x, blk=2000):
    n, c = x.shape
    s, q = pl.pallas_call(
        _stats_body,
        grid=(n // blk,),
        in_specs=[pl.BlockSpec((blk, c), lambda i: (i, 0))],
        out_specs=[pl.BlockSpec((1, c), lambda i: (0, 0))] * 2,
        out_shape=[jax.ShapeDtypeStruct((1, c), jnp.float32)] * 2,
    )(x)
    mean = s / n
    var = q / n - mean * mean
    return mean, var


def _affine(x, g, b):
    mean, var = _stats(x)
    scale = (g[None, :] / jnp.sqrt(var + 1e-5)).astype(jnp.float32)
    shift = b[None, :] - mean * scale
    return scale, shift


def _bn_relu_body(nparts, *refs):
    o_ref = refs[-1]
    col = 0
    for p in range(nparts):
        x = refs[p][...]
        s = refs[nparts + 2 * p][...]
        h = refs[nparts + 2 * p + 1][...]
        c = x.shape[1]
        o_ref[:, col:col + c] = jnp.maximum(x * s + h, 0.0)
        col += c


def _bn_relu(parts, scales, shifts, blk=2000):
    n = parts[0].shape[0]
    ctot = sum(p.shape[1] for p in parts)
    nparts = len(parts)
    in_specs = [pl.BlockSpec((blk, p.shape[1]), lambda i: (i, 0)) for p in parts]
    flat = []
    for s, h in zip(scales, shifts):
        flat += [s, h]
        in_specs += [pl.BlockSpec((1, s.shape[1]), lambda i: (0, 0))] * 2
    return pl.pallas_call(
        functools.partial(_bn_relu_body, nparts),
        grid=(n // blk,),
        in_specs=in_specs,
        out_specs=pl.BlockSpec((blk, ctot), lambda i: (i, 0)),
        out_shape=jax.ShapeDtypeStruct((n, ctot), jnp.float32),
    )(*parts, *flat)


def _mm_body(x_ref, w_ref, y_ref):
    y_ref[0] = jnp.dot(x_ref[...], w_ref[0], preferred_element_type=jnp.float32)


def _mm(x, W, blk=2000):
    n, c = x.shape
    k = W.shape[0]
    return pl.pallas_call(
        _mm_body,
        grid=(n // blk, k),
        in_specs=[
            pl.BlockSpec((blk, c), lambda i, j: (i, 0)),
            pl.BlockSpec((1, c, C_OUT), lambda i, j: (j, 0, 0)),
        ],
        out_specs=pl.BlockSpec((1, blk, C_OUT), lambda i, j: (j, i, 0)),
        out_shape=jax.ShapeDtypeStruct((k, n, C_OUT), jnp.float32),
    )(x, W)


def _res_body(t_ref, a_ref, b_ref, wa_ref, wb_ref, o_ref):
    o_ref[...] = (
        t_ref[...]
        + jnp.dot(a_ref[...], wa_ref[...], preferred_element_type=jnp.float32)
        + jnp.dot(b_ref[...], wb_ref[...], preferred_element_type=jnp.float32)
    )


def _residual(t, a, b, Wa, Wb, blk=2000):
    n = t.shape[0]
    return pl.pallas_call(
        _res_body,
        grid=(n // blk,),
        in_specs=[
            pl.BlockSpec((blk, C_OUT), lambda i: (i, 0)),
            pl.BlockSpec((blk, C_OUT), lambda i: (i, 0)),
            pl.BlockSpec((blk, C_OUT), lambda i: (i, 0)),
            pl.BlockSpec((C_OUT, C_OUT), lambda i: (0, 0)),
            pl.BlockSpec((C_OUT, C_OUT), lambda i: (0, 0)),
        ],
        out_specs=pl.BlockSpec((blk, C_OUT), lambda i: (i, 0)),
        out_shape=jax.ShapeDtypeStruct((n, C_OUT), jnp.float32),
    )(t, a, b, Wa, Wb)


NACC = 102400
ZROWS = 800
ROWS_PER_TILE = N // 16
BATCH = 128
BLK = 16


def _sc_conv_body(nblk, has_init, *refs):
    if has_init:
        yv, srcf4, dstf, init, out, acc, sidx, didx, rows, zbuf = refs
    else:
        yv, srcf4, dstf, out, acc, sidx, didx, rows, zbuf = refs
        init = None
    ci = lax.axis_index("c")
    s = lax.axis_index("s")
    eb_t = nblk * BLK

    if init is None:
        def _z(i, _):
            zbuf[i] = jnp.zeros((16,), jnp.float32)
            return 0
        lax.fori_loop(0, ZROWS, _z, 0)

    for cj in range(2):
        cc = ci * 2 + cj
        if init is None:
            for z in range(NACC // 16 // ZROWS):
                pltpu.sync_copy(zbuf, acc.at[pl.ds(s * (NACC // 16) + z * ZROWS, ZROWS)])
        else:
            pltpu.sync_copy(
                init.at[pl.ds(s * ROWS_PER_TILE, ROWS_PER_TILE), pl.ds(cc * 16, 16)],
                acc.at[pl.ds(s * ROWS_PER_TILE, ROWS_PER_TILE)],
            )
        plsc.subcore_barrier()

        def blk_body(t, _):
            row0 = s * eb_t + t * BLK
            pltpu.sync_copy(srcf4.at[cc, pl.ds(row0, BLK)], sidx)
            pltpu.sync_copy(dstf.at[pl.ds(row0, BLK)], didx)
            for jb in range(BLK):
                pltpu.sync_copy(yv.at[sidx.at[jb, 0]], rows)
                pltpu.sync_copy(rows, acc.at[didx.at[jb, 0]], add=True)
            return 0

        lax.fori_loop(0, nblk, blk_body, 0)
        plsc.subcore_barrier()

        pltpu.sync_copy(
            acc.at[pl.ds(s * ROWS_PER_TILE, ROWS_PER_TILE)],
            out.at[pl.ds(s * ROWS_PER_TILE, ROWS_PER_TILE), pl.ds(cc * 16, 16)],
        )
        if cj == 0:
            plsc.subcore_barrier()


def _sc_conv(Y, srcf4, dstf, init=None):
    k = Y.shape[0]
    eb = dstf.shape[0]
    nblk = eb // 16 // BLK
    yv = Y.reshape(k * N * 4, 16)
    mesh = plsc.VectorSubcoreMesh(core_axis_name="c", subcore_axis_name="s")
    scratch = [
        pltpu.VMEM_SHARED((NACC, 16), jnp.float32),
        pltpu.VMEM((BLK, 1, BATCH), jnp.int32),
        pltpu.VMEM((BLK, 1, BATCH), jnp.int32),
        pltpu.VMEM((BATCH, 16), jnp.float32),
        pltpu.VMEM((ZROWS, 16), jnp.float32),
    ]
    fn = pl.kernel(
        functools.partial(_sc_conv_body, nblk, init is not None),
        out_type=jax.ShapeDtypeStruct((N, C_OUT), jnp.float32),
        mesh=mesh,
        scratch_types=scratch,
        compiler_params=pltpu.CompilerParams(use_tc_tiling_on_sc=False),
    )
    args = (yv, srcf4, dstf) + ((init,) if init is not None else ())
    return fn(*args)


def _edge_indices(src, dst):
    k, e = src.shape
    etot = k * e
    epad = -(-etot // 32768) * 32768
    srcf = (src + (jnp.arange(k, dtype=jnp.int32) * N)[:, None]).reshape(-1) * 4
    dstf = dst.reshape(-1)
    srcf = jnp.concatenate([srcf, jnp.zeros((epad - etot,), jnp.int32)])
    dstf = jnp.concatenate([dstf, jnp.full((epad - etot,), N, jnp.int32)])
    srcf4 = srcf[None, :] + jnp.arange(4, dtype=jnp.int32)[:, None]
    return srcf4.reshape(4, epad // 128, 1, 128), dstf.reshape(epad // 128, 1, 128)


def kernel(feats, up_feats, inv_src, inv_dst, sub_src, sub_dst, g_up, b_up, W_inv,
           g1_0, b1_0, W1_0, g2_0, b2_0, W2_0, Wres0,
           g1_1, b1_1, W1_1, g2_1, b2_1, W2_1):
    inv_s4, inv_d = _edge_indices(inv_src, inv_dst)
    sub_s4, sub_d = _edge_indices(sub_src, sub_dst)

    sc0, sh0 = _affine(feats, g_up, b_up)
    xn = _bn_relu([feats], [sc0], [sh0])
    up = _sc_conv(_mm(xn, W_inv), inv_s4, inv_d)

    sca, sha = _affine(up_feats, g1_0[:C_OUT], b1_0[:C_OUT])
    scb, shb = _affine(up, g1_0[C_OUT:], b1_0[C_OUT:])
    xn = _bn_relu([up_feats, up], [sca, scb], [sha, shb])
    t = _sc_conv(_mm(xn, W1_0), sub_s4, sub_d)

    sc1, sh1 = _affine(t, g2_0, b2_0)
    xn = _bn_relu([t], [sc1], [sh1])
    t = _sc_conv(_mm(xn, W2_0), sub_s4, sub_d)

    h = _residual(t, up_feats, up, Wres0[:C_OUT], Wres0[C_OUT:])

    sc2, sh2 = _affine(h, g1_1, b1_1)
    xn = _bn_relu([h], [sc2], [sh2])
    t = _sc_conv(_mm(xn, W1_1), sub_s4, sub_d)

    sc3, sh3 = _affine(t, g2_1, b2_1)
    xn = _bn_relu([t], [sc3], [sh3])
    return _sc_conv(_mm(xn, W2_1), sub_s4, sub_d, init=h)


# speedup vs baseline: 1.7539x; 1.2400x over previous
import functools

import jax
import jax.numpy as jnp
from jax import lax
from jax.experimental import pallas as pl
from jax.experimental.pallas import tpu as pltpu
from jax.experimental.pallas import tpu_sc as plsc

N = 100000
C_IN = 128
C_OUT = 64
KI = 8
KS = 27


def _stats_body(x_ref, s_ref, q_ref):
    @pl.when(pl.program_id(0) == 0)
    def _init():
        s_ref[...] = jnp.zeros_like(s_ref)
        q_ref[...] = jnp.zeros_like(q_ref)

    x = x_ref[...]
    s_ref[...] += jnp.sum(x, axis=0, keepdims=True)
    q_ref[...] += jnp.sum(x * x, axis=0, keepdims=True)


def _stats(x, blk=2000):
    n, c = x.shape
    s, q = pl.pallas_call(
        _stats_body,
        grid=(n // blk,),
        in_specs=[pl.BlockSpec((blk, c), lambda i: (i, 0))],
        out_specs=[pl.BlockSpec((1, c), lambda i: (0, 0))] * 2,
        out_shape=[jax.ShapeDtypeStruct((1, c), jnp.float32)] * 2,
    )(x)
    mean = s / n
    var = q / n - mean * mean
    return mean, var


def _affine(x, g, b):
    mean, var = _stats(x)
    scale = (g[None, :] / jnp.sqrt(var + 1e-5)).astype(jnp.float32)
    shift = b[None, :] - mean * scale
    return scale, shift


def _bn_relu_body(nparts, *refs):
    o_ref = refs[-1]
    col = 0
    for p in range(nparts):
        x = refs[p][...]
        s = refs[nparts + 2 * p][...]
        h = refs[nparts + 2 * p + 1][...]
        c = x.shape[1]
        o_ref[:, col:col + c] = jnp.maximum(x * s + h, 0.0)
        col += c


def _bn_relu(parts, scales, shifts, blk=2000):
    n = parts[0].shape[0]
    ctot = sum(p.shape[1] for p in parts)
    nparts = len(parts)
    in_specs = [pl.BlockSpec((blk, p.shape[1]), lambda i: (i, 0)) for p in parts]
    flat = []
    for s, h in zip(scales, shifts):
        flat += [s, h]
        in_specs += [pl.BlockSpec((1, s.shape[1]), lambda i: (0, 0))] * 2
    return pl.pallas_call(
        functools.partial(_bn_relu_body, nparts),
        grid=(n // blk,),
        in_specs=in_specs,
        out_specs=pl.BlockSpec((blk, ctot), lambda i: (i, 0)),
        out_shape=jax.ShapeDtypeStruct((n, ctot), jnp.float32),
    )(*parts, *flat)


def _mm_body(x_ref, w_ref, y_ref):
    y_ref[0] = jnp.dot(x_ref[...], w_ref[0], preferred_element_type=jnp.float32)


def _mm(x, W, blk=2000):
    n, c = x.shape
    k = W.shape[0]
    return pl.pallas_call(
        _mm_body,
        grid=(n // blk, k),
        in_specs=[
            pl.BlockSpec((blk, c), lambda i, j: (i, 0)),
            pl.BlockSpec((1, c, C_OUT), lambda i, j: (j, 0, 0)),
        ],
        out_specs=pl.BlockSpec((1, blk, C_OUT), lambda i, j: (j, i, 0)),
        out_shape=jax.ShapeDtypeStruct((k, n, C_OUT), jnp.float32),
    )(x, W)


def _res_body(t_ref, a_ref, b_ref, wa_ref, wb_ref, o_ref):
    o_ref[...] = (
        t_ref[...]
        + jnp.dot(a_ref[...], wa_ref[...], preferred_element_type=jnp.float32)
        + jnp.dot(b_ref[...], wb_ref[...], preferred_element_type=jnp.float32)
    )


def _residual(t, a, b, Wa, Wb, blk=2000):
    n = t.shape[0]
    return pl.pallas_call(
        _res_body,
        grid=(n // blk,),
        in_specs=[
            pl.BlockSpec((blk, C_OUT), lambda i: (i, 0)),
            pl.BlockSpec((blk, C_OUT), lambda i: (i, 0)),
            pl.BlockSpec((blk, C_OUT), lambda i: (i, 0)),
            pl.BlockSpec((C_OUT, C_OUT), lambda i: (0, 0)),
            pl.BlockSpec((C_OUT, C_OUT), lambda i: (0, 0)),
        ],
        out_specs=pl.BlockSpec((blk, C_OUT), lambda i: (i, 0)),
        out_shape=jax.ShapeDtypeStruct((n, C_OUT), jnp.float32),
    )(t, a, b, Wa, Wb)


NACC = 100352
ZROWS = 224
ROWS_PER_TILE = N // 16
BATCH = 128
BLK = 8


def _sc_conv_body(nblk, has_init, *refs):
    if has_init:
        yv, srcf4, dstf, init, out, acc, sidx, didx, rows, zbuf, gsem, ssem = refs
    else:
        yv, srcf4, dstf, out, acc, sidx, didx, rows, zbuf, gsem, ssem = refs
        init = None
    ci = lax.axis_index("c")
    s = lax.axis_index("s")
    eb_t = nblk * BLK

    if init is None:
        def _z(i, _):
            zbuf[i] = jnp.zeros((16,), jnp.float32)
            return 0
        lax.fori_loop(0, ZROWS, _z, 0)

    for cj in range(2):
        cc = ci * 2 + cj
        if init is None:
            for z in range(NACC // 16 // ZROWS):
                pltpu.sync_copy(zbuf, acc.at[pl.ds(s * (NACC // 16) + z * ZROWS, ZROWS)])
        else:
            pltpu.sync_copy(
                init.at[pl.ds(s * ROWS_PER_TILE, ROWS_PER_TILE), pl.ds(cc * 16, 16)],
                acc.at[pl.ds(s * ROWS_PER_TILE, ROWS_PER_TILE)],
            )
        plsc.subcore_barrier()

        def blk_body(t, _):
            row0 = s * eb_t + t * BLK
            pltpu.sync_copy(srcf4.at[cc, pl.ds(row0, BLK)], sidx)
            pltpu.sync_copy(dstf.at[pl.ds(row0, BLK)], didx)
            gds = [pltpu.async_copy(yv.at[sidx.at[jb, 0]], rows.at[jb], gsem)
                   for jb in range(BLK)]
            for d in gds:
                d.wait()
            sds = [pltpu.async_copy(rows.at[jb], acc.at[didx.at[jb, 0]], ssem, add=True)
                   for jb in range(BLK)]
            for d in sds:
                d.wait()
            return 0

        lax.fori_loop(0, nblk, blk_body, 0)
        plsc.subcore_barrier()

        pltpu.sync_copy(
            acc.at[pl.ds(s * ROWS_PER_TILE, ROWS_PER_TILE)],
            out.at[pl.ds(s * ROWS_PER_TILE, ROWS_PER_TILE), pl.ds(cc * 16, 16)],
        )
        if cj == 0:
            plsc.subcore_barrier()


def _sc_conv(Y, srcf4, dstf, init=None):
    k = Y.shape[0]
    eb = dstf.shape[0]
    nblk = eb // 16 // BLK
    yv = Y.reshape(k * N * 4, 16)
    mesh = plsc.VectorSubcoreMesh(core_axis_name="c", subcore_axis_name="s")
    scratch = [
        pltpu.VMEM_SHARED((NACC, 16), jnp.float32),
        pltpu.VMEM((BLK, 1, BATCH), jnp.int32),
        pltpu.VMEM((BLK, 1, BATCH), jnp.int32),
        pltpu.VMEM((BLK, BATCH, 16), jnp.float32),
        pltpu.VMEM((ZROWS, 16), jnp.float32),
        pltpu.SemaphoreType.DMA,
        pltpu.SemaphoreType.DMA,
    ]
    fn = pl.kernel(
        functools.partial(_sc_conv_body, nblk, init is not None),
        out_type=jax.ShapeDtypeStruct((N, C_OUT), jnp.float32),
        mesh=mesh,
        scratch_types=scratch,
        compiler_params=pltpu.CompilerParams(use_tc_tiling_on_sc=False),
    )
    args = (yv, srcf4, dstf) + ((init,) if init is not None else ())
    return fn(*args)


def _edge_indices(src, dst):
    k, e = src.shape
    etot = k * e
    epad = -(-etot // 32768) * 32768
    srcf = (src + (jnp.arange(k, dtype=jnp.int32) * N)[:, None]).reshape(-1) * 4
    dstf = dst.reshape(-1)
    srcf = jnp.concatenate([srcf, jnp.zeros((epad - etot,), jnp.int32)])
    dstf = jnp.concatenate([dstf, jnp.full((epad - etot,), N, jnp.int32)])
    srcf4 = srcf[None, :] + jnp.arange(4, dtype=jnp.int32)[:, None]
    return srcf4.reshape(4, epad // 128, 1, 128), dstf.reshape(epad // 128, 1, 128)


def kernel(feats, up_feats, inv_src, inv_dst, sub_src, sub_dst, g_up, b_up, W_inv,
           g1_0, b1_0, W1_0, g2_0, b2_0, W2_0, Wres0,
           g1_1, b1_1, W1_1, g2_1, b2_1, W2_1):
    inv_s4, inv_d = _edge_indices(inv_src, inv_dst)
    sub_s4, sub_d = _edge_indices(sub_src, sub_dst)

    sc0, sh0 = _affine(feats, g_up, b_up)
    xn = _bn_relu([feats], [sc0], [sh0])
    up = _sc_conv(_mm(xn, W_inv), inv_s4, inv_d)

    sca, sha = _affine(up_feats, g1_0[:C_OUT], b1_0[:C_OUT])
    scb, shb = _affine(up, g1_0[C_OUT:], b1_0[C_OUT:])
    xn = _bn_relu([up_feats, up], [sca, scb], [sha, shb])
    t = _sc_conv(_mm(xn, W1_0), sub_s4, sub_d)

    sc1, sh1 = _affine(t, g2_0, b2_0)
    xn = _bn_relu([t], [sc1], [sh1])
    t = _sc_conv(_mm(xn, W2_0), sub_s4, sub_d)

    h = _residual(t, up_feats, up, Wres0[:C_OUT], Wres0[C_OUT:])

    sc2, sh2 = _affine(h, g1_1, b1_1)
    xn = _bn_relu([h], [sc2], [sh2])
    t = _sc_conv(_mm(xn, W1_1), sub_s4, sub_d)

    sc3, sh3 = _affine(t, g2_1, b2_1)
    xn = _bn_relu([t], [sc3], [sh3])
    return _sc_conv(_mm(xn, W2_1), sub_s4, sub_d, init=h)
